# baseline (speedup 1.0000x reference)
import functools

import jax
import jax.numpy as jnp
from jax import lax
from jax.experimental import pallas as pl
from jax.experimental.pallas import tpu as pltpu
from jax.experimental.pallas import tpu_sc as plsc

N = 10000
E = 160000
NT = 16
NW = 32
NQ = 4
QROWS = 2560
NP = NQ * QROWS
EPP = 160256
WPT = EPP // NW
CAPL = 1536
CAPQ = NW * CAPL
SPT = CAPQ // NT
CHQ = 32
NCHQ = SPT // CHQ
RB = 1000
RPTQ = QROWS // NT


def _make_linear(fin, fout):
    def body(x, hn8, W, b, Wh, bh, sW8, sb, out):
        xx = x[...]
        h = jnp.dot(hn8[...], sW8[...], preferred_element_type=jnp.float32) + sb[...]
        out[...] = (jnp.dot(xx, W[...], preferred_element_type=jnp.float32) + b[...]
                    + h * (jnp.dot(xx, Wh[...], preferred_element_type=jnp.float32)
                           + bh[...]))

    return pl.pallas_call(
        body,
        grid=(N // RB,),
        in_specs=[pl.BlockSpec((RB, fin), lambda r: (r, 0)),
                  pl.BlockSpec((RB, 8), lambda r: (r, 0)),
                  pl.BlockSpec((fin, fout), lambda r: (0, 0)),
                  pl.BlockSpec((1, fout), lambda r: (0, 0)),
                  pl.BlockSpec((fin, fout), lambda r: (0, 0)),
                  pl.BlockSpec((1, fout), lambda r: (0, 0)),
                  pl.BlockSpec((8, fout), lambda r: (0, 0)),
                  pl.BlockSpec((1, fout), lambda r: (0, 0))],
        out_specs=pl.BlockSpec((RB, fout), lambda r: (r, 0)),
        out_shape=jax.ShapeDtypeStruct((N, fout), jnp.float32),
    )


def _make_partition():
    mesh = plsc.VectorSubcoreMesh(core_axis_name="c", subcore_axis_name="s",
                                  num_cores=2, num_subcores=NT)
    out_type = [jax.ShapeDtypeStruct((NQ * CAPQ,), jnp.int32),
                jax.ShapeDtypeStruct((NQ * CAPQ,), jnp.int32),
                jax.ShapeDtypeStruct((NQ * CAPQ,), jnp.float32)]
    scratch_types = [
        pltpu.VMEM((WPT,), jnp.int32),
        pltpu.VMEM((WPT,), jnp.int32),
        pltpu.VMEM((WPT,), jnp.float32),
        pltpu.VMEM((NQ * (CAPL + 16),), jnp.int32),
        pltpu.VMEM((NQ * (CAPL + 16),), jnp.int32),
        pltpu.VMEM((NQ * (CAPL + 16),), jnp.float32),
    ]

    @functools.partial(pl.kernel, mesh=mesh, out_type=out_type,
                       scratch_types=scratch_types)
    def part(src_hbm, dst_hbm, ew_hbm, srcq, dstq, ewq,
             in_src, in_dst, in_ew, loc_src, loc_dst, loc_ew):
        c = lax.axis_index("c")
        s = lax.axis_index("s")
        w = c * NT + s
        ibase = w * WPT

        pltpu.sync_copy(src_hbm.at[pl.ds(ibase, WPT)], in_src)
        pltpu.sync_copy(dst_hbm.at[pl.ds(ibase, WPT)], in_dst)
        pltpu.sync_copy(ew_hbm.at[pl.ds(ibase, WPT)], in_ew)

        zi = jnp.zeros((16,), jnp.int32)
        zf = jnp.zeros((16,), jnp.float32)

        def prefill(i, carry):
            sl = pl.ds(i * 16, 16)
            loc_src[sl] = zi
            loc_dst[sl] = zi
            loc_ew[sl] = zf
            return carry

        lax.fori_loop(0, NQ * (CAPL + 16) // 16, prefill, 0)

        lane0 = lax.iota(jnp.int32, 16) == 0

        def step(g, tails):
            base = g * 16
            svw = in_src[pl.ds(base, 16)]
            dvw = in_dst[pl.ds(base, 16)]
            eww = in_ew[pl.ds(base, 16)]
            xv = lax.shift_right_logical(dvw, 9)
            qw = lax.shift_right_logical(xv * 13108, 16)
            ldw = dvw - qw * QROWS
            for ei in range(16):
                q = qw[ei]
                sel = [(q == qq).astype(jnp.int32) for qq in range(NQ)]
                t = (sel[0] * tails[0] + sel[1] * tails[1]
                     + sel[2] * tails[2] + sel[3] * tails[3])
                pos = q * (CAPL + 16) + t
                for locref, valw in ((loc_src, svw), (loc_dst, ldw),
                                     (loc_ew, eww)):
                    win = locref[pl.ds(pos, 16)]
                    locref[pl.ds(pos, 16)] = jnp.where(
                        lane0, jnp.broadcast_to(valw[ei], (16,)), win)
                tails = tuple(tails[qq] + sel[qq] for qq in range(NQ))
            return tails

        z32 = jnp.int32(0)
        lax.fori_loop(0, WPT // 16, step, (z32, z32, z32, z32))

        for qq in range(NQ):
            obase = qq * CAPQ + w * CAPL
            qb = qq * (CAPL + 16)
            pltpu.sync_copy(loc_src.at[pl.ds(qb, CAPL)],
                            srcq.at[pl.ds(obase, CAPL)])
            pltpu.sync_copy(loc_dst.at[pl.ds(qb, CAPL)],
                            dstq.at[pl.ds(obase, CAPL)])
            pltpu.sync_copy(loc_ew.at[pl.ds(qb, CAPL)],
                            ewq.at[pl.ds(obase, CAPL)])

    return part


def _make_spmm(width):
    mesh = plsc.VectorSubcoreMesh(core_axis_name="c", subcore_axis_name="s",
                                  num_cores=2, num_subcores=NT)
    U = width // 128
    out_type = [jax.ShapeDtypeStruct((NP, 128), jnp.float32) for _ in range(U)]
    scratch_types = (
        [pltpu.VMEM((SPT,), jnp.int32),
         pltpu.VMEM((CHQ,), jnp.int32),
         pltpu.VMEM((CHQ,), jnp.int32),
         pltpu.VMEM((CHQ,), jnp.float32),
         pltpu.VMEM((CHQ,), jnp.float32),
         pltpu.VMEM((CHQ, width), jnp.float32),
         pltpu.VMEM((CHQ, width), jnp.float32),
         pltpu.VMEM((CHQ, 128), jnp.float32)]
        + [pltpu.VMEM_SHARED((QROWS, 128), jnp.float32) for _ in range(U)]
        + [pltpu.SemaphoreType.DMA, pltpu.SemaphoreType.DMA]
    )

    @functools.partial(pl.kernel, mesh=mesh, out_type=out_type,
                       scratch_types=scratch_types)
    def spmm(zeros_hbm, srcq, dstq, ewq, sup, *refs):
        outs = refs[:U]
        (src_all, dst_v0, dst_v1, ew_v0, ew_v1, rows0, rows1,
         stage) = refs[U:U + 8]
        accs = refs[U + 8:2 * U + 8]
        sem0, sem1 = refs[2 * U + 8:]
        c = lax.axis_index("c")
        s = lax.axis_index("s")
        rbase = s * RPTQ
        bufs = [(dst_v0, ew_v0, rows0, sem0), (dst_v1, ew_v1, rows1, sem1)]

        def do_pass(p):
            q = 2 * c + p
            sbase = q * CAPQ + s * SPT
            pltpu.sync_copy(srcq.at[pl.ds(sbase, SPT)], src_all)
            for u in range(U):
                pltpu.sync_copy(zeros_hbm.at[pl.ds(rbase, RPTQ)],
                                accs[u].at[pl.ds(rbase, RPTQ)])
            plsc.subcore_barrier()

            def copies(j, b):
                dst_v, ew_v, rows, sem = bufs[b]
                return [
                    pltpu.make_async_copy(
                        dstq.at[pl.ds(sbase + j * CHQ, CHQ)], dst_v, sem),
                    pltpu.make_async_copy(
                        ewq.at[pl.ds(sbase + j * CHQ, CHQ)], ew_v, sem),
                    pltpu.make_async_copy(
                        sup.at[src_all.at[pl.ds(j * CHQ, CHQ)]], rows, sem),
                ]

            def fetch(j, b):
                for cp in copies(j, b):
                    cp.start()

            fetch(0, 0)

            def pair(i2, carry):
                for b in range(2):
                    j = 2 * i2 + b
                    dst_v, ew_v, rows, sem = bufs[b]

                    @pl.when(j + 1 < NCHQ)
                    def _():
                        fetch(j + 1, 1 - b)

                    for cp in copies(j, b):
                        cp.wait()

                    for u in range(U):
                        def grp(eo, c2, u=u):
                            wv = ew_v[pl.ds(eo * 16, 16)]
                            for ei in range(16):
                                e = eo * 16 + ei
                                wb = jnp.broadcast_to(wv[ei], (16,))
                                for v in range(8):
                                    ssl = pl.ds(v * 16, 16)
                                    rsl = pl.ds(u * 128 + v * 16, 16)
                                    stage[e, ssl] = rows[e, rsl] * wb
                            return c2

                        lax.fori_loop(0, CHQ // 16, grp, 0)
                        pltpu.sync_copy(stage, accs[u].at[dst_v], add=True)
                return carry

            lax.fori_loop(0, NCHQ // 2, pair, 0)
            plsc.subcore_barrier()
            for u in range(U):
                pltpu.sync_copy(accs[u].at[pl.ds(rbase, RPTQ)],
                                outs[u].at[pl.ds(q * QROWS + rbase, RPTQ)])

        do_pass(0)
        do_pass(1)

    return spmm


def _log_softmax(x):
    nfeat = x.shape[1]

    def body(xr, out):
        v = xr[...]
        m = jnp.max(v, axis=1, keepdims=True)
        ex = jnp.exp(v - m)
        lse = jnp.log(jnp.sum(ex, axis=1, keepdims=True))
        out[...] = v - m - lse

    return pl.pallas_call(
        body,
        grid=(N // RB,),
        in_specs=[pl.BlockSpec((RB, nfeat), lambda r: (r, 0))],
        out_specs=pl.BlockSpec((RB, nfeat), lambda r: (r, 0)),
        out_shape=jax.ShapeDtypeStruct((N, nfeat), jnp.float32),
    )(x)


def kernel(fea, edge_index, edge_weight, hnet_tensor, hparam_tensor,
           W0, b0, Wh0, bh0, sW0, sb0,
           W1, b1, Wh1, bh1, sW1, sb1,
           W2, b2, Wh2, bh2, sW2, sb2,
           W3, b3, Wh3, bh3, sW3, sb3):
    pad = EPP - E
    src = jnp.concatenate([edge_index[0], jnp.zeros((pad,), jnp.int32)])
    dst = jnp.concatenate([edge_index[1], jnp.zeros((pad,), jnp.int32)])
    ew = jnp.concatenate([edge_weight, jnp.zeros((pad,), jnp.float32)])
    hn8 = jnp.pad(hnet_tensor, ((0, 0), (0, 8 - hnet_tensor.shape[1])))

    srcq, dstq, ewq = _make_partition()(src, dst, ew)

    layers = [(W0, b0, Wh0, bh0, sW0, sb0),
              (W1, b1, Wh1, bh1, sW1, sb1),
              (W2, b2, Wh2, bh2, sW2, sb2),
              (W3, b3, Wh3, bh3, sW3, sb3)]

    x = fea
    for (W, b, Wh, bh, sW, sb) in layers:
        fin, fout = W.shape
        sW8 = jnp.pad(sW, ((0, 8 - sW.shape[0]), (0, 0)))
        sup = _make_linear(fin, fout)(
            x, hn8, W, b.reshape(1, -1), Wh, bh.reshape(1, -1),
            sW8, sb.reshape(1, -1))
        zeros = jnp.zeros((QROWS, 128), jnp.float32)
        outs = _make_spmm(fout)(zeros, srcq, dstq, ewq, sup)
        x = jnp.concatenate(outs, axis=1)[:N]

    return _log_softmax(x)


# speedup vs baseline: 3.5111x; 3.5111x over previous
import functools

import jax
import jax.numpy as jnp
from jax import lax
from jax.experimental import pallas as pl
from jax.experimental.pallas import tpu as pltpu
from jax.experimental.pallas import tpu_sc as plsc

N = 10000
E = 160000
FC = 128
NT = 16
NP = 10240
CH = 128
NCH = 80
EPT = CH * NCH
EP = EPT * NT
RPT = NP // NT
RB = 1000


def _linear_body(nfci, nfco, xrefs_and_rest):
    pass


def _make_linear(nfci, nfco):
    fin = nfci * FC
    fout = nfco * FC

    def body(*refs):
        xparts = refs[:nfci]
        hn8, W, b, Wh, bh, sW8, sb = refs[nfci:nfci + 7]
        outs = refs[nfci + 7:]
        xx = jnp.concatenate([p[...] for p in xparts], axis=1)
        h = jnp.dot(hn8[...], sW8[...], preferred_element_type=jnp.float32) + sb[...]
        s = (jnp.dot(xx, W[...], preferred_element_type=jnp.float32) + b[...]
             + h * (jnp.dot(xx, Wh[...], preferred_element_type=jnp.float32) + bh[...]))
        for k in range(nfco):
            outs[k][...] = s[:, k * FC:(k + 1) * FC]

    grid = (N // RB,)
    in_specs = (
        [pl.BlockSpec((RB, FC), lambda r: (r, 0)) for _ in range(nfci)]
        + [pl.BlockSpec((RB, 8), lambda r: (r, 0)),
           pl.BlockSpec((fin, fout), lambda r: (0, 0)),
           pl.BlockSpec((1, fout), lambda r: (0, 0)),
           pl.BlockSpec((fin, fout), lambda r: (0, 0)),
           pl.BlockSpec((1, fout), lambda r: (0, 0)),
           pl.BlockSpec((8, fout), lambda r: (0, 0)),
           pl.BlockSpec((1, fout), lambda r: (0, 0))]
    )
    out_specs = [pl.BlockSpec((RB, FC), lambda r: (r, 0)) for _ in range(nfco)]
    return pl.pallas_call(
        body,
        grid=grid,
        in_specs=in_specs,
        out_specs=out_specs,
        out_shape=[jax.ShapeDtypeStruct((N, FC), jnp.float32) for _ in range(nfco)],
    )


def _make_spmm():
    mesh = plsc.VectorSubcoreMesh(core_axis_name="c", subcore_axis_name="s",
                                  num_cores=2, num_subcores=NT)

    out_type = [jax.ShapeDtypeStruct((NP, FC), jnp.float32) for _ in range(2)]
    scratch_types = [
        pltpu.VMEM((EPT,), jnp.int32),
        pltpu.VMEM((CH,), jnp.int32),
        pltpu.VMEM((CH,), jnp.int32),
        pltpu.VMEM((CH,), jnp.float32),
        pltpu.VMEM((CH,), jnp.float32),
        pltpu.VMEM((CH, FC), jnp.float32),
        pltpu.VMEM((CH, FC), jnp.float32),
        pltpu.VMEM_SHARED((NP, FC), jnp.float32),
        pltpu.SemaphoreType.DMA,
        pltpu.SemaphoreType.DMA,
    ]

    @functools.partial(pl.kernel, mesh=mesh, out_type=out_type,
                       scratch_types=scratch_types)
    def spmm(zeros_hbm, src_hbm, dst_hbm, ew_hbm, sup0, sup1, out0, out1,
             src_all, dst_v0, dst_v1, ew_v0, ew_v1, rows0, rows1, acc,
             sem0, sem1):
        c = lax.axis_index("c")
        s = lax.axis_index("s")
        ebase = s * EPT
        rbase = s * RPT
        bufs = [(dst_v0, ew_v0, rows0, sem0), (dst_v1, ew_v1, rows1, sem1)]

        pltpu.sync_copy(src_hbm.at[pl.ds(ebase, EPT)], src_all)
        pltpu.sync_copy(zeros_hbm.at[pl.ds(rbase, RPT)],
                        acc.at[pl.ds(rbase, RPT)])
        plsc.subcore_barrier()

        def do_pass(sup, out):
            def copies(j, b):
                dst_v, ew_v, rows, sem = bufs[b]
                off = j * CH
                return [
                    pltpu.make_async_copy(
                        dst_hbm.at[pl.ds(ebase + off, CH)], dst_v, sem),
                    pltpu.make_async_copy(
                        ew_hbm.at[pl.ds(ebase + off, CH)], ew_v, sem),
                    pltpu.make_async_copy(
                        sup.at[src_all.at[pl.ds(off, CH)]], rows, sem),
                ]

            def fetch(j, b):
                for cp in copies(j, b):
                    cp.start()

            fetch(0, 0)

            def pair(i2, carry):
                for b in range(2):
                    j = 2 * i2 + b
                    dst_v, ew_v, rows, sem = bufs[b]

                    @pl.when(j + 1 < NCH)
                    def _():
                        fetch(j + 1, 1 - b)

                    for cp in copies(j, b):
                        cp.wait()

                    def grp(eo, c2):
                        wv = ew_v[pl.ds(eo * 16, 16)]
                        for ei in range(16):
                            e = eo * 16 + ei
                            w = jnp.broadcast_to(wv[ei], (16,))
                            for jj in range(FC // 16):
                                sl = pl.ds(jj * 16, 16)
                                rows[e, sl] = rows[e, sl] * w
                        return c2

                    lax.fori_loop(0, CH // 16, grp, 0)
                    pltpu.sync_copy(rows, acc.at[dst_v], add=True)
                return carry

            lax.fori_loop(0, NCH // 2, pair, 0)
            plsc.subcore_barrier()
            pltpu.sync_copy(acc.at[pl.ds(rbase, RPT)],
                            out.at[pl.ds(rbase, RPT)])

        @pl.when(c == 0)
        def _():
            do_pass(sup0, out0)

        @pl.when(c == 1)
        def _():
            do_pass(sup1, out1)

    return spmm


def _log_softmax(parts):
    nfc = len(parts)
    fout = nfc * FC

    def body(*refs):
        xparts = refs[:nfc]
        out = refs[nfc]
        x = jnp.concatenate([p[...] for p in xparts], axis=1)
        m = jnp.max(x, axis=1, keepdims=True)
        ex = jnp.exp(x - m)
        lse = jnp.log(jnp.sum(ex, axis=1, keepdims=True))
        out[...] = x - m - lse

    return pl.pallas_call(
        body,
        grid=(N // RB,),
        in_specs=[pl.BlockSpec((RB, FC), lambda r: (r, 0)) for _ in range(nfc)],
        out_specs=pl.BlockSpec((RB, fout), lambda r: (r, 0)),
        out_shape=jax.ShapeDtypeStruct((N, fout), jnp.float32),
    )(*parts)


def kernel(fea, edge_index, edge_weight, hnet_tensor, hparam_tensor,
           W0, b0, Wh0, bh0, sW0, sb0,
           W1, b1, Wh1, bh1, sW1, sb1,
           W2, b2, Wh2, bh2, sW2, sb2,
           W3, b3, Wh3, bh3, sW3, sb3):
    src = jnp.concatenate([edge_index[0], jnp.zeros((EP - E,), jnp.int32)])
    dst = jnp.concatenate([edge_index[1], jnp.zeros((EP - E,), jnp.int32)])
    ew = jnp.concatenate([edge_weight, jnp.zeros((EP - E,), jnp.float32)])
    zeros = jnp.zeros((NP, FC), jnp.float32)
    hn8 = jnp.pad(hnet_tensor, ((0, 0), (0, 8 - hnet_tensor.shape[1])))

    layers = [(W0, b0, Wh0, bh0, sW0, sb0),
              (W1, b1, Wh1, bh1, sW1, sb1),
              (W2, b2, Wh2, bh2, sW2, sb2),
              (W3, b3, Wh3, bh3, sW3, sb3)]

    xparts = [fea[:, 0:FC], fea[:, FC:2 * FC]]
    for (W, b, Wh, bh, sW, sb) in layers:
        nfci = W.shape[0] // FC
        nfco = W.shape[1] // FC
        sW8 = jnp.pad(sW, ((0, 8 - sW.shape[0]), (0, 0)))
        lin = _make_linear(nfci, nfco)
        sup_parts = lin(*xparts, hn8, W, b.reshape(1, -1), Wh,
                        bh.reshape(1, -1), sW8, sb.reshape(1, -1))
        spmm = _make_spmm()
        xparts = []
        for k in range(0, nfco, 2):
            o0, o1 = spmm(zeros, src, dst, ew,
                          sup_parts[k], sup_parts[k + 1])
            xparts += [o0[:N], o1[:N]]

    return _log_softmax(xparts)
